# baseline (speedup 1.0000x reference)
import dataclasses
import functools

import jax
import jax.numpy as jnp
from jax import lax
from jax.experimental import pallas as pl
from jax.experimental.pallas import tpu as pltpu
from jax.experimental.pallas import tpu_sc as plsc

N = 10000
E = 320000
D = 128
HID = 256
K = 10

NC = 2
NS = 16
L = 16
DH = D // NC
NP = 10240
RT = NP // NS
B = 128
EPT = E // NS
NB = 160
NSB = NB // 2
ETOT = NS * NB * B
CH = 160
BR = 256

_mesh = plsc.VectorSubcoreMesh(core_axis_name="c", subcore_axis_name="s")

_sc_params = pltpu.CompilerParams(
    needs_layout_passes=False, use_tc_tiling_on_sc=False)


@functools.partial(
    pl.kernel,
    out_type=jax.ShapeDtypeStruct((NP,), jnp.float32),
    mesh=_mesh,
    compiler_params=_sc_params,
    scratch_types=[
        pltpu.VMEM_SHARED((NP,), jnp.float32),
        pltpu.VMEM((NB, B), jnp.int32),
        pltpu.VMEM((B,), jnp.float32),
        pltpu.VMEM((RT,), jnp.float32),
        pltpu.SemaphoreType.DMA,
    ],
)
def _deg_kernel(dst_hbm, deg_hbm, deg_s, dst_t, ones_b, zb, dsem):
    c = lax.axis_index("c")
    s = lax.axis_index("s")
    r0 = s * RT
    pltpu.sync_copy(dst_hbm.at[s], dst_t)
    for i in range(B // L):
        ones_b[pl.ds(i * L, L)] = jnp.ones((L,), jnp.float32)

    @pl.loop(0, RT // L)
    def _(i):
        zb[pl.ds(i * L, L)] = jnp.zeros((L,), jnp.float32)

    pltpu.sync_copy(zb, deg_s.at[pl.ds(r0, RT)])
    plsc.subcore_barrier()

    @pl.loop(0, NB)
    def _(j):
        pltpu.sync_copy(ones_b, deg_s.at[dst_t.at[j]], add=True)

    plsc.subcore_barrier()

    @pl.when(c == 0)
    def _():
        pltpu.sync_copy(deg_s.at[pl.ds(r0, RT)], deg_hbm.at[pl.ds(r0, RT)])


def _mlp_body(x_ref, deg_ref, w1_ref, b1_ref, w2_ref, b2_ref, wg_ref,
              acc0_ref, w0_ref, gw0_ref, dis_ref):
    x = x_ref[...]
    h = jnp.maximum(x @ w1_ref[...] + b1_ref[...], 0.0)
    h = h @ w2_ref[...] + b2_ref[...]
    g = h @ wg_ref[...]
    deg = deg_ref[...]
    dis = jnp.where(deg > 0, lax.rsqrt(jnp.maximum(deg, 1.0)), 0.0)
    s0 = 1.0 / (1.0 + jnp.exp(-g))
    acc0_ref[...] = s0 * h
    w0_ref[...] = dis * h
    gw0_ref[...] = dis * g
    dis_ref[...] = dis


_mlp_call = pl.pallas_call(
    _mlp_body,
    grid=(NP // BR,),
    in_specs=[
        pl.BlockSpec((BR, D), lambda i: (i, 0)),
        pl.BlockSpec((BR, 1), lambda i: (i, 0)),
        pl.BlockSpec((D, HID), lambda i: (0, 0)),
        pl.BlockSpec((1, HID), lambda i: (0, 0)),
        pl.BlockSpec((HID, D), lambda i: (0, 0)),
        pl.BlockSpec((1, D), lambda i: (0, 0)),
        pl.BlockSpec((D, 1), lambda i: (0, 0)),
    ],
    out_specs=[
        pl.BlockSpec((BR, D), lambda i: (i, 0)),
        pl.BlockSpec((BR, D), lambda i: (i, 0)),
        pl.BlockSpec((BR, 1), lambda i: (i, 0)),
        pl.BlockSpec((BR, 1), lambda i: (i, 0)),
    ],
    out_shape=[
        jax.ShapeDtypeStruct((NP, D), jnp.float32),
        jax.ShapeDtypeStruct((NP, D), jnp.float32),
        jax.ShapeDtypeStruct((NP, 1), jnp.float32),
        jax.ShapeDtypeStruct((NP, 1), jnp.float32),
    ],
)


@functools.partial(
    pl.kernel,
    out_type=[
        jax.ShapeDtypeStruct((NC * NP, DH), jnp.float32),
        jax.ShapeDtypeStruct((NC * NP, DH), jnp.float32),
        jax.ShapeDtypeStruct((NC * NP,), jnp.float32),
    ],
    mesh=_mesh,
    compiler_params=_sc_params,
    scratch_types=[
        pltpu.VMEM_SHARED((NP, DH), jnp.float32),
        pltpu.VMEM_SHARED((NP,), jnp.float32),
        pltpu.VMEM((NB, B), jnp.int32),
        pltpu.VMEM((NB, B), jnp.int32),
        pltpu.VMEM((CH, DH), jnp.float32),
        pltpu.VMEM((CH, DH), jnp.float32),
        pltpu.VMEM((CH, DH), jnp.float32),
        pltpu.VMEM((CH, DH), jnp.float32),
        pltpu.VMEM((B,), jnp.float32),
        pltpu.VMEM((B,), jnp.float32),
        pltpu.VMEM((CH,), jnp.float32),
        pltpu.VMEM((CH,), jnp.float32),
        pltpu.VMEM((CH,), jnp.float32),
        pltpu.VMEM((CH,), jnp.float32),
        pltpu.VMEM((RT,), jnp.float32),
        pltpu.SemaphoreType.DMA,
        pltpu.SemaphoreType.DMA,
        pltpu.SemaphoreType.DMA,
        pltpu.SemaphoreType.DMA,
    ],
)
def _prop_kernel(acc0_hbm, w0_hbm, gw0_hbm, dis_hbm, src_hbm, dst_hbm,
                 out_hbm, w_hbm, gw_hbm,
                 t_s, gn_s, src_t, dst_t, X0, X1, abuf, zbuf,
                 gA, gB, gnb, gwb, sb, zgb, disb,
                 sgA, sgB, ssA, ssB):
    c = lax.axis_index("c")
    s = lax.axis_index("s")
    r0 = s * RT

    pltpu.sync_copy(src_hbm.at[c, s], src_t)
    pltpu.sync_copy(dst_hbm.at[s], dst_t)
    pltpu.sync_copy(dis_hbm.at[pl.ds(r0, RT)], disb)
    for i in range(CH // L):
        zgb[pl.ds(i * L, L)] = jnp.zeros((L,), jnp.float32)

    @pl.loop(0, CH)
    def _(i):
        for d in range(DH // L):
            zbuf[i, pl.ds(d * L, L)] = jnp.zeros((L,), jnp.float32)

    for cc in range(RT // CH):
        rr = r0 + cc * CH
        tgt = c * NP + rr
        pltpu.sync_copy(w0_hbm.at[pl.ds(tgt, CH)], X1)
        pltpu.sync_copy(X1, w_hbm.at[pl.ds(tgt, CH)])
        pltpu.sync_copy(acc0_hbm.at[pl.ds(tgt, CH)], abuf)
        pltpu.sync_copy(abuf, out_hbm.at[pl.ds(tgt, CH)])
        pltpu.sync_copy(gw0_hbm.at[pl.ds(rr, CH)], gwb)
        pltpu.sync_copy(gwb, gw_hbm.at[pl.ds(tgt, CH)])
        pltpu.sync_copy(zbuf, t_s.at[pl.ds(rr, CH)])
        pltpu.sync_copy(zgb, gn_s.at[pl.ds(rr, CH)])
    plsc.subcore_barrier()

    def issue_g(j, X, gb, sem):
        pltpu.async_copy(w_hbm.at[src_t.at[j]], X.at[pl.ds(0, B)], sem)
        pltpu.async_copy(gw_hbm.at[src_t.at[j]], gb, sem)

    def wait_g(X, gb, sem):
        pltpu.make_async_copy(w_hbm.at[src_t.at[0]], X.at[pl.ds(0, B)],
                              sem).wait()
        pltpu.make_async_copy(gw_hbm.at[src_t.at[0]], gb, sem).wait()

    def issue_s(j, X, gb, sem):
        pltpu.async_copy(X.at[pl.ds(0, B)], t_s.at[dst_t.at[j]], sem,
                         add=True)
        pltpu.async_copy(gb, gn_s.at[dst_t.at[j]], sem, add=True)

    def wait_s(X, gb, sem):
        pltpu.make_async_copy(X.at[pl.ds(0, B)], t_s.at[dst_t.at[0]],
                              sem).wait()
        pltpu.make_async_copy(gb, gn_s.at[dst_t.at[0]], sem).wait()

    @pl.loop(0, K)
    def _(_k):
        issue_g(0, X0, gA, sgA)
        wait_g(X0, gA, sgA)
        issue_s(0, X0, gA, ssA)
        issue_g(1, X1, gB, sgB)
        wait_s(X0, gA, ssA)
        issue_g(2, X0, gA, sgA)
        wait_g(X1, gB, sgB)
        issue_s(1, X1, gB, ssB)

        @pl.loop(1, NB // 2 - 1)
        def _(m):
            j0 = 2 * m
            j1 = 2 * m + 1
            wait_g(X0, gA, sgA)
            issue_s(j0, X0, gA, ssA)
            wait_s(X1, gB, ssB)
            issue_g(j1, X1, gB, sgB)
            wait_s(X0, gA, ssA)
            issue_g(j0 + 2, X0, gA, sgA)
            wait_g(X1, gB, sgB)
            issue_s(j1, X1, gB, ssB)

        wait_g(X0, gA, sgA)
        issue_s(NB - 2, X0, gA, ssA)
        wait_s(X1, gB, ssB)
        issue_g(NB - 1, X1, gB, sgB)
        wait_s(X0, gA, ssA)
        wait_g(X1, gB, sgB)
        issue_s(NB - 1, X1, gB, ssB)
        wait_s(X1, gB, ssB)

        plsc.subcore_barrier()

        @pl.loop(0, RT // CH)
        def _(cc):
            rr = r0 + cc * CH
            tgt = c * NP + rr
            pltpu.sync_copy(t_s.at[pl.ds(rr, CH)], X0)
            pltpu.sync_copy(zbuf, t_s.at[pl.ds(rr, CH)])
            pltpu.sync_copy(gn_s.at[pl.ds(rr, CH)], gnb)
            pltpu.sync_copy(zgb, gn_s.at[pl.ds(rr, CH)])
            pltpu.sync_copy(out_hbm.at[pl.ds(tgt, CH)], abuf)
            for i in range(CH // L):
                sl = pl.ds(i * L, L)
                dv = disb[pl.ds(cc * CH + i * L, L)]
                gv = dv * gnb[sl]
                sb[sl] = 1.0 / (1.0 + jnp.exp(-gv))
                gwb[sl] = dv * gv
            pltpu.sync_copy(gwb, gw_hbm.at[pl.ds(tgt, CH)])

            @pl.loop(0, CH)
            def _(i):
                dvb = plsc.load_gather(
                    disb, [jnp.full((L,), cc * CH + i, jnp.int32)])
                svb = plsc.load_gather(sb, [jnp.full((L,), i, jnp.int32)])
                for d in range(DH // L):
                    dd = pl.ds(d * L, L)
                    hv = dvb * X0[i, dd]
                    abuf[i, dd] += svb * hv
                    X1[i, dd] = dvb * hv

            pltpu.sync_copy(abuf, out_hbm.at[pl.ds(tgt, CH)])
            pltpu.sync_copy(X1, w_hbm.at[pl.ds(tgt, CH)])

        plsc.subcore_barrier()


@jax.jit
def _run(features, edge_index, W1, b1, W2, b2, w_gate):
    src = edge_index[0].astype(jnp.int32)
    dst = edge_index[1].astype(jnp.int32)
    pad = ETOT - E
    srcp = jnp.concatenate([src, jnp.full((pad,), N, jnp.int32)])
    dstp = jnp.concatenate([dst, jnp.full((pad,), N, jnp.int32)])
    srcp = srcp.reshape(NS, NB, B)
    dstp = dstp.reshape(NS, NB, B)
    srco = jnp.stack([srcp, srcp + NP])

    xp = jnp.zeros((NP, D), jnp.float32).at[:N].set(features)

    deg = _deg_kernel(dstp)
    acc0, w0, gw0, dis = _mlp_call(
        xp, deg.reshape(NP, 1), W1, b1.reshape(1, HID), W2,
        b2.reshape(1, D), w_gate)
    acc0f = acc0.reshape(NP, NC, DH).swapaxes(0, 1).reshape(NC * NP, DH)
    w0f = w0.reshape(NP, NC, DH).swapaxes(0, 1).reshape(NC * NP, DH)
    out, _, _ = _prop_kernel(acc0f, w0f, gw0.reshape(NP), dis.reshape(NP),
                             srco, dstp)
    return out.reshape(NC, NP, DH).swapaxes(0, 1).reshape(NP, D)[:N]


def kernel(features, edge_index, W1, b1, W2, b2, w_gate):
    return _run(features, edge_index, W1, b1, W2, b2, w_gate)


# speedup vs baseline: 1.2460x; 1.2460x over previous
import dataclasses
import functools

import jax
import jax.numpy as jnp
from jax import lax
from jax.experimental import pallas as pl
from jax.experimental.pallas import tpu as pltpu
from jax.experimental.pallas import tpu_sc as plsc

N = 10000
E = 320000
D = 128
HID = 256
K = 10

NC = 2
NS = 16
L = 16
DH = D // NC
NP = 10240
RT = NP // NS
B = 128
EPT = E // NS
NB = 159
ETOT = NS * NB * B
CH = 128
BR = 256

_mesh = plsc.VectorSubcoreMesh(core_axis_name="c", subcore_axis_name="s")

_sc_params = pltpu.CompilerParams(
    needs_layout_passes=False, use_tc_tiling_on_sc=False)


@functools.partial(
    pl.kernel,
    out_type=jax.ShapeDtypeStruct((NP,), jnp.float32),
    mesh=_mesh,
    compiler_params=_sc_params,
    scratch_types=[
        pltpu.VMEM_SHARED((NP,), jnp.float32),
        pltpu.VMEM((NB, B), jnp.int32),
        pltpu.VMEM((B,), jnp.float32),
        pltpu.VMEM((RT,), jnp.float32),
        pltpu.SemaphoreType.DMA,
    ],
)
def _deg_kernel(dst_hbm, deg_hbm, deg_s, dst_t, ones_b, zb, dsem):
    c = lax.axis_index("c")
    s = lax.axis_index("s")
    r0 = s * RT
    pltpu.sync_copy(dst_hbm.at[s], dst_t)
    for i in range(B // L):
        ones_b[pl.ds(i * L, L)] = jnp.ones((L,), jnp.float32)

    @pl.loop(0, RT // L)
    def _(i):
        zb[pl.ds(i * L, L)] = jnp.zeros((L,), jnp.float32)

    pltpu.sync_copy(zb, deg_s.at[pl.ds(r0, RT)])
    plsc.subcore_barrier()

    @pl.loop(0, NB)
    def _(j):
        pltpu.sync_copy(ones_b, deg_s.at[dst_t.at[j]], add=True)

    plsc.subcore_barrier()

    @pl.when(c == 0)
    def _():
        pltpu.sync_copy(deg_s.at[pl.ds(r0, RT)], deg_hbm.at[pl.ds(r0, RT)])


def _mlp_body(x_ref, deg_ref, w1_ref, b1_ref, w2_ref, b2_ref, wg_ref,
              acc0_ref, w0_ref, gw0_ref, dis_ref):
    x = x_ref[...]
    h = jnp.maximum(x @ w1_ref[...] + b1_ref[...], 0.0)
    h = h @ w2_ref[...] + b2_ref[...]
    g = h @ wg_ref[...]
    deg = deg_ref[...]
    dis = jnp.where(deg > 0, lax.rsqrt(jnp.maximum(deg, 1.0)), 0.0)
    s0 = 1.0 / (1.0 + jnp.exp(-g))
    acc0_ref[...] = s0 * h
    w0_ref[...] = dis * h
    gw0_ref[...] = dis * g
    dis_ref[...] = dis


_mlp_call = pl.pallas_call(
    _mlp_body,
    grid=(NP // BR,),
    in_specs=[
        pl.BlockSpec((BR, D), lambda i: (i, 0)),
        pl.BlockSpec((BR, 1), lambda i: (i, 0)),
        pl.BlockSpec((D, HID), lambda i: (0, 0)),
        pl.BlockSpec((1, HID), lambda i: (0, 0)),
        pl.BlockSpec((HID, D), lambda i: (0, 0)),
        pl.BlockSpec((1, D), lambda i: (0, 0)),
        pl.BlockSpec((D, 1), lambda i: (0, 0)),
    ],
    out_specs=[
        pl.BlockSpec((BR, D), lambda i: (i, 0)),
        pl.BlockSpec((BR, D), lambda i: (i, 0)),
        pl.BlockSpec((BR, 1), lambda i: (i, 0)),
        pl.BlockSpec((BR, 1), lambda i: (i, 0)),
    ],
    out_shape=[
        jax.ShapeDtypeStruct((NP, D), jnp.float32),
        jax.ShapeDtypeStruct((NP, D), jnp.float32),
        jax.ShapeDtypeStruct((NP, 1), jnp.float32),
        jax.ShapeDtypeStruct((NP, 1), jnp.float32),
    ],
)


@functools.partial(
    pl.kernel,
    out_type=[
        jax.ShapeDtypeStruct((NC * NP, DH), jnp.float32),
        jax.ShapeDtypeStruct((NC * NP, DH), jnp.float32),
        jax.ShapeDtypeStruct((NC * NP,), jnp.float32),
    ],
    mesh=_mesh,
    compiler_params=_sc_params,
    scratch_types=[
        pltpu.VMEM_SHARED((NP, DH), jnp.float32),
        pltpu.VMEM_SHARED((NP,), jnp.float32),
        pltpu.VMEM((NB, B), jnp.int32),
        pltpu.VMEM((NB, B), jnp.int32),
        pltpu.VMEM((CH, DH), jnp.float32),
        pltpu.VMEM((CH, DH), jnp.float32),
        pltpu.VMEM((CH, DH), jnp.float32),
        pltpu.VMEM((CH, DH), jnp.float32),
        pltpu.VMEM((CH, DH), jnp.float32),
        pltpu.VMEM((B,), jnp.float32),
        pltpu.VMEM((B,), jnp.float32),
        pltpu.VMEM((B,), jnp.float32),
        pltpu.VMEM((CH,), jnp.float32),
        pltpu.VMEM((CH,), jnp.float32),
        pltpu.VMEM((CH,), jnp.float32),
        pltpu.VMEM((CH,), jnp.float32),
        pltpu.VMEM((RT,), jnp.float32),
        pltpu.SemaphoreType.DMA,
        pltpu.SemaphoreType.DMA,
        pltpu.SemaphoreType.DMA,
        pltpu.SemaphoreType.DMA,
        pltpu.SemaphoreType.DMA,
        pltpu.SemaphoreType.DMA,
    ],
)
def _prop_kernel(acc0_hbm, w0_hbm, gw0_hbm, dis_hbm, src_hbm, dst_hbm,
                 out_hbm, w_hbm, gw_hbm,
                 t_s, gn_s, src_t, dst_t, X0, X1, X2, abuf, zbuf,
                 gA, gB, gC, gnb, gwb, sb, zgb, disb,
                 sgA, sgB, sgC, ssA, ssB, ssC):
    c = lax.axis_index("c")
    s = lax.axis_index("s")
    r0 = s * RT

    pltpu.sync_copy(src_hbm.at[c, s], src_t)
    pltpu.sync_copy(dst_hbm.at[s], dst_t)
    pltpu.sync_copy(dis_hbm.at[pl.ds(r0, RT)], disb)
    for i in range(CH // L):
        zgb[pl.ds(i * L, L)] = jnp.zeros((L,), jnp.float32)

    @pl.loop(0, CH)
    def _(i):
        for d in range(DH // L):
            zbuf[i, pl.ds(d * L, L)] = jnp.zeros((L,), jnp.float32)

    for cc in range(RT // CH):
        rr = r0 + cc * CH
        tgt = c * NP + rr
        pltpu.sync_copy(w0_hbm.at[pl.ds(tgt, CH)], X1)
        pltpu.sync_copy(X1, w_hbm.at[pl.ds(tgt, CH)])
        pltpu.sync_copy(acc0_hbm.at[pl.ds(tgt, CH)], abuf)
        pltpu.sync_copy(abuf, out_hbm.at[pl.ds(tgt, CH)])
        pltpu.sync_copy(gw0_hbm.at[pl.ds(rr, CH)], gwb)
        pltpu.sync_copy(gwb, gw_hbm.at[pl.ds(tgt, CH)])
        pltpu.sync_copy(zbuf, t_s.at[pl.ds(rr, CH)])
        pltpu.sync_copy(zgb, gn_s.at[pl.ds(rr, CH)])
    plsc.subcore_barrier()

    def issue_g(j, X, gb, sem):
        pltpu.async_copy(w_hbm.at[src_t.at[j]], X.at[pl.ds(0, B)], sem)
        pltpu.async_copy(gw_hbm.at[src_t.at[j]], gb, sem)

    def wait_g(X, gb, sem):
        pltpu.make_async_copy(w_hbm.at[src_t.at[0]], X.at[pl.ds(0, B)],
                              sem).wait()
        pltpu.make_async_copy(gw_hbm.at[src_t.at[0]], gb, sem).wait()

    def issue_s(j, X, gb, sem):
        pltpu.async_copy(X.at[pl.ds(0, B)], t_s.at[dst_t.at[j]], sem,
                         add=True)
        pltpu.async_copy(gb, gn_s.at[dst_t.at[j]], sem, add=True)

    def wait_s(X, gb, sem):
        pltpu.make_async_copy(X.at[pl.ds(0, B)], t_s.at[dst_t.at[0]],
                              sem).wait()
        pltpu.make_async_copy(gb, gn_s.at[dst_t.at[0]], sem).wait()

    bufs = ((X0, gA, sgA, ssA), (X1, gB, sgB, ssB), (X2, gC, sgC, ssC))

    @pl.loop(0, K)
    def _(_k):
        for b, (X, gb, sg, ss) in enumerate(bufs):
            issue_g(b, X, gb, sg)
        for b, (X, gb, sg, ss) in enumerate(bufs):
            wait_g(X, gb, sg)
            issue_s(b, X, gb, ss)
        for b, (X, gb, sg, ss) in enumerate(bufs):
            wait_s(X, gb, ss)
            issue_g(b + 3, X, gb, sg)

        @pl.loop(1, NB // 3 - 1)
        def _(t):
            j = 3 * t
            for b, (X, gb, sg, ss) in enumerate(bufs):
                wait_g(X, gb, sg)
                issue_s(j + b, X, gb, ss)
            for b, (X, gb, sg, ss) in enumerate(bufs):
                wait_s(X, gb, ss)
                issue_g(j + b + 3, X, gb, sg)

        for b, (X, gb, sg, ss) in enumerate(bufs):
            wait_g(X, gb, sg)
            issue_s(NB - 3 + b, X, gb, ss)
        for b, (X, gb, sg, ss) in enumerate(bufs):
            wait_s(X, gb, ss)

        plsc.subcore_barrier()

        @pl.loop(0, RT // CH)
        def _(cc):
            rr = r0 + cc * CH
            tgt = c * NP + rr
            pltpu.sync_copy(t_s.at[pl.ds(rr, CH)], X0)
            pltpu.sync_copy(zbuf, t_s.at[pl.ds(rr, CH)])
            pltpu.sync_copy(gn_s.at[pl.ds(rr, CH)], gnb)
            pltpu.sync_copy(zgb, gn_s.at[pl.ds(rr, CH)])
            pltpu.sync_copy(out_hbm.at[pl.ds(tgt, CH)], abuf)
            for i in range(CH // L):
                sl = pl.ds(i * L, L)
                dv = disb[pl.ds(cc * CH + i * L, L)]
                gv = dv * gnb[sl]
                sb[sl] = 1.0 / (1.0 + jnp.exp(-gv))
                gwb[sl] = dv * gv
            pltpu.sync_copy(gwb, gw_hbm.at[pl.ds(tgt, CH)])

            @pl.loop(0, CH)
            def _(i):
                dvb = plsc.load_gather(
                    disb, [jnp.full((L,), cc * CH + i, jnp.int32)])
                svb = plsc.load_gather(sb, [jnp.full((L,), i, jnp.int32)])
                for d in range(DH // L):
                    dd = pl.ds(d * L, L)
                    hv = dvb * X0[i, dd]
                    abuf[i, dd] += svb * hv
                    X1[i, dd] = dvb * hv

            pltpu.sync_copy(abuf, out_hbm.at[pl.ds(tgt, CH)])
            pltpu.sync_copy(X1, w_hbm.at[pl.ds(tgt, CH)])

        plsc.subcore_barrier()


@jax.jit
def _run(features, edge_index, W1, b1, W2, b2, w_gate):
    src = edge_index[0].astype(jnp.int32)
    dst = edge_index[1].astype(jnp.int32)
    pad = ETOT - E
    srcp = jnp.concatenate([src, jnp.full((pad,), N, jnp.int32)])
    dstp = jnp.concatenate([dst, jnp.full((pad,), N, jnp.int32)])
    srcp = srcp.reshape(NS, NB, B)
    dstp = dstp.reshape(NS, NB, B)
    srco = jnp.stack([srcp, srcp + NP])

    xp = jnp.zeros((NP, D), jnp.float32).at[:N].set(features)

    deg = _deg_kernel(dstp)
    acc0, w0, gw0, dis = _mlp_call(
        xp, deg.reshape(NP, 1), W1, b1.reshape(1, HID), W2,
        b2.reshape(1, D), w_gate)
    acc0f = acc0.reshape(NP, NC, DH).swapaxes(0, 1).reshape(NC * NP, DH)
    w0f = w0.reshape(NP, NC, DH).swapaxes(0, 1).reshape(NC * NP, DH)
    out, _, _ = _prop_kernel(acc0f, w0f, gw0.reshape(NP), dis.reshape(NP),
                             srco, dstp)
    return out.reshape(NC, NP, DH).swapaxes(0, 1).reshape(NP, D)[:N]


def kernel(features, edge_index, W1, b1, W2, b2, w_gate):
    return _run(features, edge_index, W1, b1, W2, b2, w_gate)
